# baseline (speedup 1.0000x reference)
import functools

import jax
import jax.numpy as jnp
from jax import lax
from jax.experimental import pallas as pl
from jax.experimental.pallas import tpu as pltpu
from jax.experimental.pallas import tpu_sc as plsc

EPS = 1e-5
TILE = 128


def _prep_body(x_ref, w_ref, o_ref):
    o_ref[...] = jnp.dot(x_ref[...], w_ref[...],
                         preferred_element_type=jnp.float32)


def _prep(x, w1xt):
    n, d = x.shape
    blk = 1000
    return pl.pallas_call(
        _prep_body,
        grid=(n // blk,),
        in_specs=[pl.BlockSpec((blk, d), lambda i: (i, 0)),
                  pl.BlockSpec(w1xt.shape, lambda i: (0, 0))],
        out_specs=pl.BlockSpec((blk, 128), lambda i: (i, 0)),
        out_shape=jax.ShapeDtypeStruct((n, 128), jnp.float32),
    )(x, w1xt)


def _sc_gather(p0, col, ea, n_edges, n_pad):
    info = plsc.get_sparse_core_info()
    nc, ns = info.num_cores, info.num_subcores
    nw = nc * ns
    nt = n_edges // TILE
    iters = (nt + nw - 1) // nw
    rows_per_tile = n_pad // ns
    sub = 128
    nsub = rows_per_tile // sub
    mesh = plsc.VectorSubcoreMesh(core_axis_name="c", subcore_axis_name="s")

    @functools.partial(
        pl.kernel, mesh=mesh,
        compiler_params=pltpu.CompilerParams(use_tc_tiling_on_sc=False),
        out_type=(jax.ShapeDtypeStruct((n_edges, 128), jnp.float32),
                  jax.ShapeDtypeStruct((nc * n_pad, 16), jnp.float32),
                  jax.ShapeDtypeStruct((nc * n_pad, 16), jnp.float32)),
        scratch_types=[
            pltpu.VMEM((TILE,), jnp.int32),
            pltpu.VMEM((TILE,), jnp.int32),
            pltpu.VMEM((TILE, 128), jnp.float32),
            pltpu.VMEM((TILE, 16), jnp.float32),
            pltpu.VMEM((TILE, 16), jnp.float32),
            pltpu.VMEM((sub, 16), jnp.float32),
            pltpu.VMEM_SHARED((n_pad, 16), jnp.float32),
            pltpu.VMEM_SHARED((n_pad, 16), jnp.float32),
            pltpu.SemaphoreType.DMA,
        ],
    )
    def k(p0_hbm, col_hbm, ea_hbm, out_hbm, cnt_out, aseg_out,
          idx_v, ramp_v, rows_v, eab_v, ones_v, zb_v, cnt_sh, aseg_sh, sem):
        cid = lax.axis_index("c")
        sid = lax.axis_index("s")
        wid = sid * nc + cid

        zeros16 = jnp.zeros((16,), jnp.float32)
        ones16 = jnp.ones((16,), jnp.float32)
        iota16 = lax.iota(jnp.int32, 16)

        def zr(i, carry):
            zb_v[i, :] = zeros16
            return carry

        lax.fori_loop(0, sub, zr, 0)

        def orow(i, carry):
            ones_v[i, :] = ones16
            return carry

        lax.fori_loop(0, TILE, orow, 0)

        def set_ramp(r0):
            def rg(g, carry):
                ramp_v[pl.ds(g * 16, 16)] = iota16 + (r0 + g * 16)
                return carry

            lax.fori_loop(0, TILE // 16, rg, 0)

        def zs(kk, carry):
            r0 = sid * rows_per_tile + kk * sub
            set_ramp(r0)
            pltpu.sync_copy(zb_v, cnt_sh.at[ramp_v])
            pltpu.sync_copy(zb_v, aseg_sh.at[ramp_v])
            return carry

        lax.fori_loop(0, nsub, zs, 0)
        plsc.subcore_barrier()

        def body(i, carry):
            t = wid + i * nw

            @pl.when(t < nt)
            def _():
                base = t * TILE
                pltpu.sync_copy(col_hbm.at[pl.ds(base, TILE)], idx_v)
                pltpu.async_copy(p0_hbm.at[idx_v], rows_v, sem).wait()
                pltpu.sync_copy(rows_v, out_hbm.at[pl.ds(base, TILE)])
                pltpu.sync_copy(ea_hbm.at[pl.ds(base, TILE)], eab_v)
                pltpu.sync_copy(eab_v, aseg_sh.at[idx_v], add=True)
                pltpu.sync_copy(ones_v, cnt_sh.at[idx_v], add=True)

            return carry

        lax.fori_loop(0, iters, body, 0)
        plsc.subcore_barrier()

        def wout(kk, carry):
            r0 = sid * rows_per_tile + kk * sub
            set_ramp(r0)
            pltpu.sync_copy(cnt_sh.at[ramp_v], zb_v)
            pltpu.sync_copy(zb_v, cnt_out.at[pl.ds(cid * n_pad + r0, sub)])
            pltpu.sync_copy(aseg_sh.at[ramp_v], zb_v)
            pltpu.sync_copy(zb_v, aseg_out.at[pl.ds(cid * n_pad + r0, sub)])
            return carry

        lax.fori_loop(0, nsub, wout, 0)

    return k(p0, col, ea)


def _eastats_body(a_ref, o_ref, s_acc, g_acc):
    i = pl.program_id(0)

    @pl.when(i == 0)
    def _():
        s_acc[...] = jnp.zeros_like(s_acc)
        g_acc[...] = jnp.zeros_like(g_acc)

    a = a_ref[...]
    s_acc[...] += jnp.sum(a, axis=0, keepdims=True)
    g_acc[...] += jnp.dot(a.T, a, preferred_element_type=jnp.float32)

    @pl.when(i == pl.num_programs(0) - 1)
    def _():
        o_ref[0:1, :] = s_acc[...]
        o_ref[1:17, :] = g_acc[...]


def _eastats(ea, be):
    e = ea.shape[0]
    return pl.pallas_call(
        _eastats_body,
        grid=(e // be,),
        in_specs=[pl.BlockSpec((be, 16), lambda i: (i, 0))],
        out_specs=pl.BlockSpec((17, 16), lambda i: (0, 0)),
        out_shape=jax.ShapeDtypeStruct((17, 16), jnp.float32),
        scratch_shapes=[pltpu.VMEM((1, 16), jnp.float32),
                        pltpu.VMEM((16, 16), jnp.float32)],
    )(ea)


def _comb_body(p_ref, cnt_ref, aseg_ref, eas_ref, w1a_ref, o_ref,
               *, n, n_pad):
    cnt = cnt_ref[0:n, 0:1] + cnt_ref[n_pad:n_pad + n, 0:1]
    aseg = aseg_ref[0:n, :] + aseg_ref[n_pad:n_pad + n, :]
    p0 = p_ref[...]
    w1a = w1a_ref[...]
    sa = eas_ref[0:1, :]
    gaa = eas_ref[1:17, :]
    b = jnp.dot(aseg, w1a, preferred_element_type=jnp.float32)
    s1 = (jnp.sum(cnt * p0, axis=0, keepdims=True)
          + jnp.dot(sa, w1a, preferred_element_type=jnp.float32))
    q1 = (jnp.sum(cnt * p0 * p0, axis=0, keepdims=True)
          + 2.0 * jnp.sum(p0 * b, axis=0, keepdims=True)
          + jnp.sum(w1a * jnp.dot(gaa, w1a,
                                  preferred_element_type=jnp.float32),
                    axis=0, keepdims=True))
    o_ref[0:1, :] = s1
    o_ref[1:2, :] = q1


def _comb(p0, cnt2, aseg2, eas, w1at, n_pad):
    n = p0.shape[0]
    return pl.pallas_call(
        functools.partial(_comb_body, n=n, n_pad=n_pad),
        in_specs=[pl.BlockSpec(p0.shape, lambda: (0, 0)),
                  pl.BlockSpec(cnt2.shape, lambda: (0, 0)),
                  pl.BlockSpec(aseg2.shape, lambda: (0, 0)),
                  pl.BlockSpec(eas.shape, lambda: (0, 0)),
                  pl.BlockSpec(w1at.shape, lambda: (0, 0))],
        out_specs=pl.BlockSpec((2, 128), lambda: (0, 0)),
        out_shape=jax.ShapeDtypeStruct((2, 128), jnp.float32),
    )(p0, cnt2, aseg2, eas, w1at)


def _pass2_body(p_ref, a_ref, w1a_ref, w2_ref, s1_ref, gb1_ref,
                y2a_ref, y2b_ref, o_ref, s_acc, q_acc, *, n_edges):
    i = pl.program_id(0)

    @pl.when(i == 0)
    def _():
        s_acc[...] = jnp.zeros_like(s_acc)
        q_acc[...] = jnp.zeros_like(q_acc)

    m1 = s1_ref[0:1, :] / n_edges
    v1 = s1_ref[1:2, :] / n_edges - m1 * m1
    a1 = gb1_ref[0:1, :] * lax.rsqrt(v1 + EPS)
    c1 = gb1_ref[1:2, :] - m1 * a1
    y1 = p_ref[...] + jnp.dot(a_ref[...], w1a_ref[...],
                              preferred_element_type=jnp.float32)
    z1 = jnp.maximum(y1 * a1 + c1, 0.0)
    y2 = jnp.dot(z1, w2_ref[...], preferred_element_type=jnp.float32)
    y2a_ref[...] = y2[:, 0:64]
    y2b_ref[...] = y2[:, 64:128]
    s_acc[...] += jnp.sum(y2, axis=0, keepdims=True)
    q_acc[...] += jnp.sum(y2 * y2, axis=0, keepdims=True)

    @pl.when(i == pl.num_programs(0) - 1)
    def _():
        o_ref[0:1, :] = s_acc[...]
        o_ref[1:2, :] = q_acc[...]


def _pass2(p0g, ea, w1at, w2t, s1q1, gb1, be):
    e = p0g.shape[0]
    return pl.pallas_call(
        functools.partial(_pass2_body, n_edges=float(e)),
        grid=(e // be,),
        in_specs=[pl.BlockSpec((be, 128), lambda i: (i, 0)),
                  pl.BlockSpec((be, 16), lambda i: (i, 0)),
                  pl.BlockSpec((16, 128), lambda i: (0, 0)),
                  pl.BlockSpec((128, 128), lambda i: (0, 0)),
                  pl.BlockSpec((2, 128), lambda i: (0, 0)),
                  pl.BlockSpec((2, 128), lambda i: (0, 0))],
        out_specs=[pl.BlockSpec((be, 64), lambda i: (i, 0)),
                   pl.BlockSpec((be, 64), lambda i: (i, 0)),
                   pl.BlockSpec((2, 128), lambda i: (0, 0))],
        out_shape=[jax.ShapeDtypeStruct((e, 64), jnp.float32),
                   jax.ShapeDtypeStruct((e, 64), jnp.float32),
                   jax.ShapeDtypeStruct((2, 128), jnp.float32)],
        scratch_shapes=[pltpu.VMEM((1, 128), jnp.float32),
                        pltpu.VMEM((1, 128), jnp.float32)],
    )(p0g, ea, w1at, w2t, s1q1, gb1)


def _sc_scatter(y2a, y2b, row, n_pad):
    info = plsc.get_sparse_core_info()
    nc, ns = info.num_cores, info.num_subcores
    n_edges = y2a.shape[0]
    nt = n_edges // TILE
    iters = (nt + ns - 1) // ns
    rows_per_tile = n_pad // ns
    sub = 128
    nsub = rows_per_tile // sub
    mesh = plsc.VectorSubcoreMesh(core_axis_name="c", subcore_axis_name="s")

    @functools.partial(
        pl.kernel, mesh=mesh,
        compiler_params=pltpu.CompilerParams(use_tc_tiling_on_sc=False),
        out_type=(jax.ShapeDtypeStruct((nc * n_pad, 64), jnp.float32),
                  jax.ShapeDtypeStruct((n_pad, 16), jnp.float32)),
        scratch_types=[
            pltpu.VMEM((TILE,), jnp.int32),
            pltpu.VMEM((TILE,), jnp.int32),
            pltpu.VMEM((TILE, 64), jnp.float32),
            pltpu.VMEM((TILE, 16), jnp.float32),
            pltpu.VMEM((sub, 64), jnp.float32),
            pltpu.VMEM((sub, 16), jnp.float32),
            pltpu.VMEM_SHARED((n_pad, 64), jnp.float32),
            pltpu.VMEM_SHARED((n_pad, 16), jnp.float32),
        ],
    )
    def k(y2a_hbm, y2b_hbm, row_hbm, z_out, c_out,
          idx_v, ramp_v, val_v, ones_v, zb_v, cb_v, z_sh, c_sh):
        cid = lax.axis_index("c")
        sid = lax.axis_index("s")

        zeros16 = jnp.zeros((16,), jnp.float32)
        ones16 = jnp.ones((16,), jnp.float32)

        def zrow(i, carry):
            def zcol(j, c2):
                zb_v[i, pl.ds(j * 16, 16)] = zeros16
                return c2

            lax.fori_loop(0, 4, zcol, 0)
            cb_v[i, :] = zeros16
            return carry

        lax.fori_loop(0, sub, zrow, 0)

        def orow(i, carry):
            ones_v[i, :] = ones16
            return carry

        lax.fori_loop(0, TILE, orow, 0)

        iota16 = lax.iota(jnp.int32, 16)

        def set_ramp(r0):
            def rg(g, carry):
                ramp_v[pl.ds(g * 16, 16)] = iota16 + (r0 + g * 16)
                return carry

            lax.fori_loop(0, TILE // 16, rg, 0)

        def zs(kk, carry):
            r0 = sid * rows_per_tile + kk * sub
            set_ramp(r0)
            pltpu.sync_copy(zb_v, z_sh.at[ramp_v])
            pltpu.sync_copy(cb_v, c_sh.at[ramp_v])
            return carry

        lax.fori_loop(0, nsub, zs, 0)
        plsc.subcore_barrier()

        def main_loop(src_hbm, with_count):
            def body(i, carry):
                t = sid + i * ns

                @pl.when(t < nt)
                def _():
                    base = t * TILE
                    pltpu.sync_copy(row_hbm.at[pl.ds(base, TILE)], idx_v)
                    pltpu.sync_copy(src_hbm.at[pl.ds(base, TILE)], val_v)
                    pltpu.sync_copy(val_v, z_sh.at[idx_v], add=True)
                    if with_count:
                        pltpu.sync_copy(ones_v, c_sh.at[idx_v], add=True)

                return carry

            lax.fori_loop(0, iters, body, 0)

        @pl.when(cid == 0)
        def _():
            main_loop(y2a_hbm, True)

        @pl.when(cid == 1)
        def _():
            main_loop(y2b_hbm, False)

        plsc.subcore_barrier()

        def wout(kk, carry):
            r0 = sid * rows_per_tile + kk * sub
            set_ramp(r0)
            pltpu.sync_copy(z_sh.at[ramp_v], zb_v)
            pltpu.sync_copy(zb_v, z_out.at[pl.ds(cid * n_pad + r0, sub)])

            @pl.when(cid == 0)
            def _():
                pltpu.sync_copy(c_sh.at[ramp_v], cb_v)
                pltpu.sync_copy(cb_v, c_out.at[pl.ds(r0, sub)])

            return carry

        lax.fori_loop(0, nsub, wout, 0)

    return k(y2a, y2b, row)


def _nodeA_body(x_ref, z0_ref, z1_ref, c0_ref, u_ref, b_ref,
                s2_ref, gb2_ref, w3_ref, y3_ref, o_ref, s_acc, q_acc,
                *, n_edges, n_groups):
    i = pl.program_id(0)

    @pl.when(i == 0)
    def _():
        s_acc[...] = jnp.zeros_like(s_acc)
        q_acc[...] = jnp.zeros_like(q_acc)

    m2 = s2_ref[0:1, :] / n_edges
    v2 = s2_ref[1:2, :] / n_edges - m2 * m2
    a2 = gb2_ref[0:1, :] * lax.rsqrt(v2 + EPS)
    c2 = gb2_ref[1:2, :] - m2 * a2
    zsum = jnp.concatenate([z0_ref[...], z1_ref[...]], axis=1)
    cnt = c0_ref[:, 0:1]
    agg = (zsum * a2 + cnt * c2) / jnp.maximum(cnt, 1.0)
    blk = b_ref.shape[0]
    oh = (lax.broadcasted_iota(jnp.int32, (blk, n_groups), 1)
          == b_ref[...]).astype(jnp.float32)
    ub = jnp.dot(oh, u_ref[...], preferred_element_type=jnp.float32)
    y3 = (jnp.dot(x_ref[...], w3_ref[0:128, :],
                  preferred_element_type=jnp.float32)
          + jnp.dot(agg, w3_ref[128:256, :],
                    preferred_element_type=jnp.float32)
          + jnp.dot(ub, w3_ref[256:384, :],
                    preferred_element_type=jnp.float32))
    y3_ref[...] = y3
    s_acc[...] += jnp.sum(y3, axis=0, keepdims=True)
    q_acc[...] += jnp.sum(y3 * y3, axis=0, keepdims=True)

    @pl.when(i == pl.num_programs(0) - 1)
    def _():
        o_ref[0:1, :] = s_acc[...]
        o_ref[1:2, :] = q_acc[...]


def _nodeA(x, z2a, z2b, c2a, u, batch2, s2q2, gb2, w3t, n_edges, blk):
    n = x.shape[0]
    g = u.shape[0]
    nblk = n // blk
    return pl.pallas_call(
        functools.partial(_nodeA_body, n_edges=float(n_edges), n_groups=g),
        grid=(nblk,),
        in_specs=[pl.BlockSpec((blk, 128), lambda i: (i, 0)),
                  pl.BlockSpec((blk, 64), lambda i: (i, 0)),
                  pl.BlockSpec((blk, 64), lambda i: (i, 0)),
                  pl.BlockSpec((blk, 16), lambda i: (i, 0)),
                  pl.BlockSpec((g, 128), lambda i: (0, 0)),
                  pl.BlockSpec((blk, 1), lambda i: (i, 0)),
                  pl.BlockSpec((2, 128), lambda i: (0, 0)),
                  pl.BlockSpec((2, 128), lambda i: (0, 0)),
                  pl.BlockSpec((384, 128), lambda i: (0, 0))],
        out_specs=[pl.BlockSpec((blk, 128), lambda i: (i, 0)),
                   pl.BlockSpec((2, 128), lambda i: (0, 0))],
        out_shape=[jax.ShapeDtypeStruct((n, 128), jnp.float32),
                   jax.ShapeDtypeStruct((2, 128), jnp.float32)],
        scratch_shapes=[pltpu.VMEM((1, 128), jnp.float32),
                        pltpu.VMEM((1, 128), jnp.float32)],
    )(x, z2a, z2b, c2a, u, batch2, s2q2, gb2, w3t)


def _nodeB_body(y3_ref, s3_ref, gb3_ref, w4_ref, y4_ref, o_ref,
                s_acc, q_acc, *, n_nodes):
    i = pl.program_id(0)

    @pl.when(i == 0)
    def _():
        s_acc[...] = jnp.zeros_like(s_acc)
        q_acc[...] = jnp.zeros_like(q_acc)

    m3 = s3_ref[0:1, :] / n_nodes
    v3 = s3_ref[1:2, :] / n_nodes - m3 * m3
    a3 = gb3_ref[0:1, :] * lax.rsqrt(v3 + EPS)
    c3 = gb3_ref[1:2, :] - m3 * a3
    z3 = jnp.maximum(y3_ref[...] * a3 + c3, 0.0)
    y4 = jnp.dot(z3, w4_ref[...], preferred_element_type=jnp.float32)
    y4_ref[...] = y4
    s_acc[...] += jnp.sum(y4, axis=0, keepdims=True)
    q_acc[...] += jnp.sum(y4 * y4, axis=0, keepdims=True)

    @pl.when(i == pl.num_programs(0) - 1)
    def _():
        o_ref[0:1, :] = s_acc[...]
        o_ref[1:2, :] = q_acc[...]


def _nodeB(y3, s3q3, gb3, w4t, blk):
    n = y3.shape[0]
    return pl.pallas_call(
        functools.partial(_nodeB_body, n_nodes=float(n)),
        grid=(n // blk,),
        in_specs=[pl.BlockSpec((blk, 128), lambda i: (i, 0)),
                  pl.BlockSpec((2, 128), lambda i: (0, 0)),
                  pl.BlockSpec((2, 128), lambda i: (0, 0)),
                  pl.BlockSpec((128, 128), lambda i: (0, 0))],
        out_specs=[pl.BlockSpec((blk, 128), lambda i: (i, 0)),
                   pl.BlockSpec((2, 128), lambda i: (0, 0))],
        out_shape=[jax.ShapeDtypeStruct((n, 128), jnp.float32),
                   jax.ShapeDtypeStruct((2, 128), jnp.float32)],
        scratch_shapes=[pltpu.VMEM((1, 128), jnp.float32),
                        pltpu.VMEM((1, 128), jnp.float32)],
    )(y3, s3q3, gb3, w4t)


def _nodeC_body(y4_ref, s4_ref, gb4_ref, o_ref, *, n_nodes):
    m4 = s4_ref[0:1, :] / n_nodes
    v4 = s4_ref[1:2, :] / n_nodes - m4 * m4
    a4 = gb4_ref[0:1, :] * lax.rsqrt(v4 + EPS)
    c4 = gb4_ref[1:2, :] - m4 * a4
    o_ref[...] = y4_ref[...] * a4 + c4


def _nodeC(y4, s4q4, gb4, blk):
    n = y4.shape[0]
    return pl.pallas_call(
        functools.partial(_nodeC_body, n_nodes=float(n)),
        grid=(n // blk,),
        in_specs=[pl.BlockSpec((blk, 128), lambda i: (i, 0)),
                  pl.BlockSpec((2, 128), lambda i: (0, 0)),
                  pl.BlockSpec((2, 128), lambda i: (0, 0))],
        out_specs=pl.BlockSpec((blk, 128), lambda i: (i, 0)),
        out_shape=jax.ShapeDtypeStruct((n, 128), jnp.float32),
    )(y4, s4q4, gb4)


def kernel(x, edge_index, edge_attr, u, batch,
           W1, g1, b1, W2, g2, b2, W3, g3, b3, W4, g4, b4):
    n, d_node = x.shape
    e = edge_attr.shape[0]

    row = edge_index[0]
    col = edge_index[1]
    w1xt = jnp.transpose(W1[:, :d_node])
    w1at = jnp.transpose(W1[:, d_node:])
    w2t = jnp.transpose(W2)
    w3t = jnp.transpose(W3)
    w4t = jnp.transpose(W4)
    gb1 = jnp.stack([g1, b1])
    gb2 = jnp.stack([g2, b2])
    gb3 = jnp.stack([g3, b3])
    gb4 = jnp.stack([g4, b4])
    batch2 = batch.reshape(n, 1)

    n_pad = 10240
    p0 = _prep(x, w1xt)
    p0g, cnt2, aseg2 = _sc_gather(p0, col, edge_attr, e, n_pad)
    eas = _eastats(edge_attr, be=8000)
    s1q1 = _comb(p0, cnt2, aseg2, eas, w1at, n_pad)
    y2a, y2b, s2q2 = _pass2(p0g, edge_attr, w1at, w2t, s1q1, gb1, be=2560)
    z2, c2 = _sc_scatter(y2a, y2b, row, n_pad)
    z2a, z2b = z2[0:n], z2[n_pad:n_pad + n]
    c2a = c2[0:n]
    y3, s3q3 = _nodeA(x, z2a, z2b, c2a, u, batch2, s2q2, gb2, w3t, e,
                      blk=2000)
    y4, s4q4 = _nodeB(y3, s3q3, gb3, w4t, blk=2000)
    out = _nodeC(y4, s4q4, gb4, blk=2000)
    return out


# speedup vs baseline: 1.2693x; 1.2693x over previous
import functools

import jax
import jax.numpy as jnp
from jax import lax
from jax.experimental import pallas as pl
from jax.experimental.pallas import tpu as pltpu
from jax.experimental.pallas import tpu_sc as plsc

EPS = 1e-5
TILE = 128


def _prep_body(x_ref, w_ref, o_ref):
    o_ref[...] = jnp.dot(x_ref[...], w_ref[...],
                         preferred_element_type=jnp.float32)


def _prep(x, w1xt):
    n, d = x.shape
    blk = 1000
    return pl.pallas_call(
        _prep_body,
        grid=(n // blk,),
        in_specs=[pl.BlockSpec((blk, d), lambda i: (i, 0)),
                  pl.BlockSpec(w1xt.shape, lambda i: (0, 0))],
        out_specs=pl.BlockSpec((blk, 128), lambda i: (i, 0)),
        out_shape=jax.ShapeDtypeStruct((n, 128), jnp.float32),
    )(x, w1xt)


def _sc_gather(p0, col, ea, n_edges, n_pad):
    info = plsc.get_sparse_core_info()
    nc, ns = info.num_cores, info.num_subcores
    nw = nc * ns
    nt = n_edges // TILE
    iters = (nt + nw - 1) // nw
    rows_per_tile = n_pad // ns
    sub = 128
    nsub = rows_per_tile // sub
    mesh = plsc.VectorSubcoreMesh(core_axis_name="c", subcore_axis_name="s")

    @functools.partial(
        pl.kernel, mesh=mesh,
        compiler_params=pltpu.CompilerParams(use_tc_tiling_on_sc=False),
        out_type=(jax.ShapeDtypeStruct((n_edges, 128), jnp.float32),
                  jax.ShapeDtypeStruct((nc * n_pad, 16), jnp.float32),
                  jax.ShapeDtypeStruct((nc * n_pad, 16), jnp.float32)),
        scratch_types=[
            pltpu.VMEM((TILE,), jnp.int32),
            pltpu.VMEM((TILE,), jnp.int32),
            pltpu.VMEM((TILE,), jnp.int32),
            pltpu.VMEM((TILE, 128), jnp.float32),
            pltpu.VMEM((TILE, 128), jnp.float32),
            pltpu.VMEM((TILE, 16), jnp.float32),
            pltpu.VMEM((TILE, 16), jnp.float32),
            pltpu.VMEM((TILE, 16), jnp.float32),
            pltpu.VMEM((sub, 16), jnp.float32),
            pltpu.VMEM_SHARED((n_pad, 16), jnp.float32),
            pltpu.VMEM_SHARED((n_pad, 16), jnp.float32),
            pltpu.SemaphoreType.DMA,
            pltpu.SemaphoreType.DMA,
            pltpu.SemaphoreType.DMA,
            pltpu.SemaphoreType.DMA,
        ],
    )
    def k(p0_hbm, col_hbm, ea_hbm, out_hbm, cnt_out, aseg_out,
          idx0_v, idx1_v, ramp_v, rows0_v, rows1_v, eab0_v, eab1_v,
          ones_v, zb_v, cnt_sh, aseg_sh, semL0, semL1, semG0, semG1):
        cid = lax.axis_index("c")
        sid = lax.axis_index("s")
        wid = sid * nc + cid

        zeros16 = jnp.zeros((16,), jnp.float32)
        ones16 = jnp.ones((16,), jnp.float32)
        iota16 = lax.iota(jnp.int32, 16)

        def zr(i, carry):
            zb_v[i, :] = zeros16
            return carry

        lax.fori_loop(0, sub, zr, 0)

        def orow(i, carry):
            ones_v[i, :] = ones16
            return carry

        lax.fori_loop(0, TILE, orow, 0)

        def set_ramp(r0):
            def rg(g, carry):
                ramp_v[pl.ds(g * 16, 16)] = iota16 + (r0 + g * 16)
                return carry

            lax.fori_loop(0, TILE // 16, rg, 0)

        def zs(kk, carry):
            r0 = sid * rows_per_tile + kk * sub
            set_ramp(r0)
            pltpu.sync_copy(zb_v, cnt_sh.at[ramp_v])
            pltpu.sync_copy(zb_v, aseg_sh.at[ramp_v])
            return carry

        lax.fori_loop(0, nsub, zs, 0)
        plsc.subcore_barrier()

        bufs = ((idx0_v, rows0_v, eab0_v, semL0, semG0),
                (idx1_v, rows1_v, eab1_v, semL1, semG1))

        def issue_loads(i, b):
            idx_b, _, ea_b, semL_b, _ = bufs[b]
            t = wid + i * nw

            @pl.when(t < nt)
            def _():
                base = t * TILE
                pltpu.async_copy(col_hbm.at[pl.ds(base, TILE)], idx_b, semL_b)
                pltpu.async_copy(ea_hbm.at[pl.ds(base, TILE)], ea_b, semL_b)

        def drain_loads(i, b):
            idx_b, _, ea_b, semL_b, _ = bufs[b]
            t = wid + i * nw

            @pl.when(t < nt)
            def _():
                base = t * TILE
                pltpu.make_async_copy(
                    col_hbm.at[pl.ds(base, TILE)], idx_b, semL_b).wait()
                pltpu.make_async_copy(
                    ea_hbm.at[pl.ds(base, TILE)], ea_b, semL_b).wait()

        def issue_gather(i, b):
            idx_b, rows_b, _, _, semG_b = bufs[b]
            t = wid + i * nw

            @pl.when(t < nt)
            def _():
                pltpu.async_copy(p0_hbm.at[idx_b], rows_b, semG_b)

        def process(i, b):
            idx_b, rows_b, ea_b, _, semG_b = bufs[b]
            t = wid + i * nw

            @pl.when((t >= 0) & (t < nt))
            def _():
                base = t * TILE
                pltpu.make_async_copy(
                    p0_hbm.at[idx_b], rows_b, semG_b).wait()
                pltpu.sync_copy(rows_b, out_hbm.at[pl.ds(base, TILE)])
                pltpu.sync_copy(ea_b, aseg_sh.at[idx_b], add=True)
                pltpu.sync_copy(ones_v, cnt_sh.at[idx_b], add=True)

        issue_loads(0, 0)

        def body(step, carry):
            for b in (0, 1):
                i = 2 * step + b
                drain_loads(i, b)
                issue_gather(i, b)
                process(i - 1, 1 - b)
                issue_loads(i + 1, 1 - b)
            return carry

        pairs = (iters + 1) // 2
        lax.fori_loop(0, pairs, body, 0)
        process(2 * pairs - 1, 1)
        plsc.subcore_barrier()

        def wout(kk, carry):
            r0 = sid * rows_per_tile + kk * sub
            set_ramp(r0)
            pltpu.sync_copy(cnt_sh.at[ramp_v], zb_v)
            pltpu.sync_copy(zb_v, cnt_out.at[pl.ds(cid * n_pad + r0, sub)])
            pltpu.sync_copy(aseg_sh.at[ramp_v], zb_v)
            pltpu.sync_copy(zb_v, aseg_out.at[pl.ds(cid * n_pad + r0, sub)])
            return carry

        lax.fori_loop(0, nsub, wout, 0)

    return k(p0, col, ea)


def _eastats_body(a_ref, o_ref, s_acc, g_acc):
    i = pl.program_id(0)

    @pl.when(i == 0)
    def _():
        s_acc[...] = jnp.zeros_like(s_acc)
        g_acc[...] = jnp.zeros_like(g_acc)

    a = a_ref[...]
    s_acc[...] += jnp.sum(a, axis=0, keepdims=True)
    g_acc[...] += jnp.dot(a.T, a, preferred_element_type=jnp.float32)

    @pl.when(i == pl.num_programs(0) - 1)
    def _():
        o_ref[0:1, :] = s_acc[...]
        o_ref[1:17, :] = g_acc[...]


def _eastats(ea, be):
    e = ea.shape[0]
    return pl.pallas_call(
        _eastats_body,
        grid=(e // be,),
        in_specs=[pl.BlockSpec((be, 16), lambda i: (i, 0))],
        out_specs=pl.BlockSpec((17, 16), lambda i: (0, 0)),
        out_shape=jax.ShapeDtypeStruct((17, 16), jnp.float32),
        scratch_shapes=[pltpu.VMEM((1, 16), jnp.float32),
                        pltpu.VMEM((16, 16), jnp.float32)],
    )(ea)


def _comb_body(p_ref, cnt_ref, aseg_ref, eas_ref, w1a_ref, o_ref,
               *, n, n_pad):
    cnt = cnt_ref[0:n, 0:1] + cnt_ref[n_pad:n_pad + n, 0:1]
    aseg = aseg_ref[0:n, :] + aseg_ref[n_pad:n_pad + n, :]
    p0 = p_ref[...]
    w1a = w1a_ref[...]
    sa = eas_ref[0:1, :]
    gaa = eas_ref[1:17, :]
    b = jnp.dot(aseg, w1a, preferred_element_type=jnp.float32)
    s1 = (jnp.sum(cnt * p0, axis=0, keepdims=True)
          + jnp.dot(sa, w1a, preferred_element_type=jnp.float32))
    q1 = (jnp.sum(cnt * p0 * p0, axis=0, keepdims=True)
          + 2.0 * jnp.sum(p0 * b, axis=0, keepdims=True)
          + jnp.sum(w1a * jnp.dot(gaa, w1a,
                                  preferred_element_type=jnp.float32),
                    axis=0, keepdims=True))
    o_ref[0:1, :] = s1
    o_ref[1:2, :] = q1


def _comb(p0, cnt2, aseg2, eas, w1at, n_pad):
    n = p0.shape[0]
    return pl.pallas_call(
        functools.partial(_comb_body, n=n, n_pad=n_pad),
        in_specs=[pl.BlockSpec(p0.shape, lambda: (0, 0)),
                  pl.BlockSpec(cnt2.shape, lambda: (0, 0)),
                  pl.BlockSpec(aseg2.shape, lambda: (0, 0)),
                  pl.BlockSpec(eas.shape, lambda: (0, 0)),
                  pl.BlockSpec(w1at.shape, lambda: (0, 0))],
        out_specs=pl.BlockSpec((2, 128), lambda: (0, 0)),
        out_shape=jax.ShapeDtypeStruct((2, 128), jnp.float32),
    )(p0, cnt2, aseg2, eas, w1at)


def _pass2_body(p_ref, a_ref, w1a_ref, w2_ref, s1_ref, gb1_ref,
                y2a_ref, y2b_ref, o_ref, s_acc, q_acc, *, n_edges):
    i = pl.program_id(0)

    @pl.when(i == 0)
    def _():
        s_acc[...] = jnp.zeros_like(s_acc)
        q_acc[...] = jnp.zeros_like(q_acc)

    m1 = s1_ref[0:1, :] / n_edges
    v1 = s1_ref[1:2, :] / n_edges - m1 * m1
    a1 = gb1_ref[0:1, :] * lax.rsqrt(v1 + EPS)
    c1 = gb1_ref[1:2, :] - m1 * a1
    y1 = p_ref[...] + jnp.dot(a_ref[...], w1a_ref[...],
                              preferred_element_type=jnp.float32)
    z1 = jnp.maximum(y1 * a1 + c1, 0.0)
    y2 = jnp.dot(z1, w2_ref[...], preferred_element_type=jnp.float32)
    y2a_ref[...] = y2[:, 0:64]
    y2b_ref[...] = y2[:, 64:128]
    s_acc[...] += jnp.sum(y2, axis=0, keepdims=True)
    q_acc[...] += jnp.sum(y2 * y2, axis=0, keepdims=True)

    @pl.when(i == pl.num_programs(0) - 1)
    def _():
        o_ref[0:1, :] = s_acc[...]
        o_ref[1:2, :] = q_acc[...]


def _pass2(p0g, ea, w1at, w2t, s1q1, gb1, be):
    e = p0g.shape[0]
    return pl.pallas_call(
        functools.partial(_pass2_body, n_edges=float(e)),
        grid=(e // be,),
        in_specs=[pl.BlockSpec((be, 128), lambda i: (i, 0)),
                  pl.BlockSpec((be, 16), lambda i: (i, 0)),
                  pl.BlockSpec((16, 128), lambda i: (0, 0)),
                  pl.BlockSpec((128, 128), lambda i: (0, 0)),
                  pl.BlockSpec((2, 128), lambda i: (0, 0)),
                  pl.BlockSpec((2, 128), lambda i: (0, 0))],
        out_specs=[pl.BlockSpec((be, 64), lambda i: (i, 0)),
                   pl.BlockSpec((be, 64), lambda i: (i, 0)),
                   pl.BlockSpec((2, 128), lambda i: (0, 0))],
        out_shape=[jax.ShapeDtypeStruct((e, 64), jnp.float32),
                   jax.ShapeDtypeStruct((e, 64), jnp.float32),
                   jax.ShapeDtypeStruct((2, 128), jnp.float32)],
        scratch_shapes=[pltpu.VMEM((1, 128), jnp.float32),
                        pltpu.VMEM((1, 128), jnp.float32)],
    )(p0g, ea, w1at, w2t, s1q1, gb1)


def _sc_scatter(y2a, y2b, row, n_pad):
    info = plsc.get_sparse_core_info()
    nc, ns = info.num_cores, info.num_subcores
    n_edges = y2a.shape[0]
    nt = n_edges // TILE
    iters = (nt + ns - 1) // ns
    rows_per_tile = n_pad // ns
    sub = 128
    nsub = rows_per_tile // sub
    mesh = plsc.VectorSubcoreMesh(core_axis_name="c", subcore_axis_name="s")

    @functools.partial(
        pl.kernel, mesh=mesh,
        compiler_params=pltpu.CompilerParams(use_tc_tiling_on_sc=False),
        out_type=(jax.ShapeDtypeStruct((nc * n_pad, 64), jnp.float32),
                  jax.ShapeDtypeStruct((n_pad, 16), jnp.float32)),
        scratch_types=[
            pltpu.VMEM((TILE,), jnp.int32),
            pltpu.VMEM((TILE,), jnp.int32),
            pltpu.VMEM((TILE,), jnp.int32),
            pltpu.VMEM((TILE, 64), jnp.float32),
            pltpu.VMEM((TILE, 64), jnp.float32),
            pltpu.VMEM((TILE, 16), jnp.float32),
            pltpu.VMEM((sub, 64), jnp.float32),
            pltpu.VMEM((sub, 16), jnp.float32),
            pltpu.VMEM_SHARED((n_pad, 64), jnp.float32),
            pltpu.VMEM_SHARED((n_pad, 16), jnp.float32),
            pltpu.SemaphoreType.DMA,
            pltpu.SemaphoreType.DMA,
        ],
    )
    def k(y2a_hbm, y2b_hbm, row_hbm, z_out, c_out,
          idx0_v, idx1_v, ramp_v, val0_v, val1_v, ones_v, zb_v, cb_v,
          z_sh, c_sh, semL0, semL1):
        cid = lax.axis_index("c")
        sid = lax.axis_index("s")

        zeros16 = jnp.zeros((16,), jnp.float32)
        ones16 = jnp.ones((16,), jnp.float32)

        def zrow(i, carry):
            def zcol(j, c2):
                zb_v[i, pl.ds(j * 16, 16)] = zeros16
                return c2

            lax.fori_loop(0, 4, zcol, 0)
            cb_v[i, :] = zeros16
            return carry

        lax.fori_loop(0, sub, zrow, 0)

        def orow(i, carry):
            ones_v[i, :] = ones16
            return carry

        lax.fori_loop(0, TILE, orow, 0)

        iota16 = lax.iota(jnp.int32, 16)

        def set_ramp(r0):
            def rg(g, carry):
                ramp_v[pl.ds(g * 16, 16)] = iota16 + (r0 + g * 16)
                return carry

            lax.fori_loop(0, TILE // 16, rg, 0)

        def zs(kk, carry):
            r0 = sid * rows_per_tile + kk * sub
            set_ramp(r0)
            pltpu.sync_copy(zb_v, z_sh.at[ramp_v])
            pltpu.sync_copy(cb_v, c_sh.at[ramp_v])
            return carry

        lax.fori_loop(0, nsub, zs, 0)
        plsc.subcore_barrier()

        def main_loop(src_hbm, with_count):
            bufs = ((idx0_v, val0_v, semL0), (idx1_v, val1_v, semL1))

            def issue_loads(i, b):
                idx_b, val_b, semL_b = bufs[b]
                t = sid + i * ns

                @pl.when(t < nt)
                def _():
                    base = t * TILE
                    pltpu.async_copy(
                        row_hbm.at[pl.ds(base, TILE)], idx_b, semL_b)
                    pltpu.async_copy(
                        src_hbm.at[pl.ds(base, TILE)], val_b, semL_b)

            def process(i, b):
                idx_b, val_b, semL_b = bufs[b]
                t = sid + i * ns

                @pl.when(t < nt)
                def _():
                    base = t * TILE
                    pltpu.make_async_copy(
                        row_hbm.at[pl.ds(base, TILE)], idx_b, semL_b).wait()
                    pltpu.make_async_copy(
                        src_hbm.at[pl.ds(base, TILE)], val_b, semL_b).wait()
                    pltpu.sync_copy(val_b, z_sh.at[idx_b], add=True)
                    if with_count:
                        pltpu.sync_copy(ones_v, c_sh.at[idx_b], add=True)

            issue_loads(0, 0)

            def body(step, carry):
                for b in (0, 1):
                    i = 2 * step + b
                    issue_loads(i + 1, 1 - b)
                    process(i, b)
                return carry

            pairs = (iters + 1) // 2
            lax.fori_loop(0, pairs, body, 0)

        @pl.when(cid == 0)
        def _():
            main_loop(y2a_hbm, True)

        @pl.when(cid == 1)
        def _():
            main_loop(y2b_hbm, False)

        plsc.subcore_barrier()

        def wout(kk, carry):
            r0 = sid * rows_per_tile + kk * sub
            set_ramp(r0)
            pltpu.sync_copy(z_sh.at[ramp_v], zb_v)
            pltpu.sync_copy(zb_v, z_out.at[pl.ds(cid * n_pad + r0, sub)])

            @pl.when(cid == 0)
            def _():
                pltpu.sync_copy(c_sh.at[ramp_v], cb_v)
                pltpu.sync_copy(cb_v, c_out.at[pl.ds(r0, sub)])

            return carry

        lax.fori_loop(0, nsub, wout, 0)

    return k(y2a, y2b, row)


def _nodeA_body(x_ref, z0_ref, z1_ref, c0_ref, u_ref, b_ref,
                s2_ref, gb2_ref, w3_ref, y3_ref, o_ref, s_acc, q_acc,
                *, n_edges, n_groups):
    i = pl.program_id(0)

    @pl.when(i == 0)
    def _():
        s_acc[...] = jnp.zeros_like(s_acc)
        q_acc[...] = jnp.zeros_like(q_acc)

    m2 = s2_ref[0:1, :] / n_edges
    v2 = s2_ref[1:2, :] / n_edges - m2 * m2
    a2 = gb2_ref[0:1, :] * lax.rsqrt(v2 + EPS)
    c2 = gb2_ref[1:2, :] - m2 * a2
    zsum = jnp.concatenate([z0_ref[...], z1_ref[...]], axis=1)
    cnt = c0_ref[:, 0:1]
    agg = (zsum * a2 + cnt * c2) / jnp.maximum(cnt, 1.0)
    blk = b_ref.shape[0]
    oh = (lax.broadcasted_iota(jnp.int32, (blk, n_groups), 1)
          == b_ref[...]).astype(jnp.float32)
    ub = jnp.dot(oh, u_ref[...], preferred_element_type=jnp.float32)
    y3 = (jnp.dot(x_ref[...], w3_ref[0:128, :],
                  preferred_element_type=jnp.float32)
          + jnp.dot(agg, w3_ref[128:256, :],
                    preferred_element_type=jnp.float32)
          + jnp.dot(ub, w3_ref[256:384, :],
                    preferred_element_type=jnp.float32))
    y3_ref[...] = y3
    s_acc[...] += jnp.sum(y3, axis=0, keepdims=True)
    q_acc[...] += jnp.sum(y3 * y3, axis=0, keepdims=True)

    @pl.when(i == pl.num_programs(0) - 1)
    def _():
        o_ref[0:1, :] = s_acc[...]
        o_ref[1:2, :] = q_acc[...]


def _nodeA(x, z2a, z2b, c2a, u, batch2, s2q2, gb2, w3t, n_edges, blk):
    n = x.shape[0]
    g = u.shape[0]
    nblk = n // blk
    return pl.pallas_call(
        functools.partial(_nodeA_body, n_edges=float(n_edges), n_groups=g),
        grid=(nblk,),
        in_specs=[pl.BlockSpec((blk, 128), lambda i: (i, 0)),
                  pl.BlockSpec((blk, 64), lambda i: (i, 0)),
                  pl.BlockSpec((blk, 64), lambda i: (i, 0)),
                  pl.BlockSpec((blk, 16), lambda i: (i, 0)),
                  pl.BlockSpec((g, 128), lambda i: (0, 0)),
                  pl.BlockSpec((blk, 1), lambda i: (i, 0)),
                  pl.BlockSpec((2, 128), lambda i: (0, 0)),
                  pl.BlockSpec((2, 128), lambda i: (0, 0)),
                  pl.BlockSpec((384, 128), lambda i: (0, 0))],
        out_specs=[pl.BlockSpec((blk, 128), lambda i: (i, 0)),
                   pl.BlockSpec((2, 128), lambda i: (0, 0))],
        out_shape=[jax.ShapeDtypeStruct((n, 128), jnp.float32),
                   jax.ShapeDtypeStruct((2, 128), jnp.float32)],
        scratch_shapes=[pltpu.VMEM((1, 128), jnp.float32),
                        pltpu.VMEM((1, 128), jnp.float32)],
    )(x, z2a, z2b, c2a, u, batch2, s2q2, gb2, w3t)


def _nodeB_body(y3_ref, s3_ref, gb3_ref, w4_ref, y4_ref, o_ref,
                s_acc, q_acc, *, n_nodes):
    i = pl.program_id(0)

    @pl.when(i == 0)
    def _():
        s_acc[...] = jnp.zeros_like(s_acc)
        q_acc[...] = jnp.zeros_like(q_acc)

    m3 = s3_ref[0:1, :] / n_nodes
    v3 = s3_ref[1:2, :] / n_nodes - m3 * m3
    a3 = gb3_ref[0:1, :] * lax.rsqrt(v3 + EPS)
    c3 = gb3_ref[1:2, :] - m3 * a3
    z3 = jnp.maximum(y3_ref[...] * a3 + c3, 0.0)
    y4 = jnp.dot(z3, w4_ref[...], preferred_element_type=jnp.float32)
    y4_ref[...] = y4
    s_acc[...] += jnp.sum(y4, axis=0, keepdims=True)
    q_acc[...] += jnp.sum(y4 * y4, axis=0, keepdims=True)

    @pl.when(i == pl.num_programs(0) - 1)
    def _():
        o_ref[0:1, :] = s_acc[...]
        o_ref[1:2, :] = q_acc[...]


def _nodeB(y3, s3q3, gb3, w4t, blk):
    n = y3.shape[0]
    return pl.pallas_call(
        functools.partial(_nodeB_body, n_nodes=float(n)),
        grid=(n // blk,),
        in_specs=[pl.BlockSpec((blk, 128), lambda i: (i, 0)),
                  pl.BlockSpec((2, 128), lambda i: (0, 0)),
                  pl.BlockSpec((2, 128), lambda i: (0, 0)),
                  pl.BlockSpec((128, 128), lambda i: (0, 0))],
        out_specs=[pl.BlockSpec((blk, 128), lambda i: (i, 0)),
                   pl.BlockSpec((2, 128), lambda i: (0, 0))],
        out_shape=[jax.ShapeDtypeStruct((n, 128), jnp.float32),
                   jax.ShapeDtypeStruct((2, 128), jnp.float32)],
        scratch_shapes=[pltpu.VMEM((1, 128), jnp.float32),
                        pltpu.VMEM((1, 128), jnp.float32)],
    )(y3, s3q3, gb3, w4t)


def _nodeC_body(y4_ref, s4_ref, gb4_ref, o_ref, *, n_nodes):
    m4 = s4_ref[0:1, :] / n_nodes
    v4 = s4_ref[1:2, :] / n_nodes - m4 * m4
    a4 = gb4_ref[0:1, :] * lax.rsqrt(v4 + EPS)
    c4 = gb4_ref[1:2, :] - m4 * a4
    o_ref[...] = y4_ref[...] * a4 + c4


def _nodeC(y4, s4q4, gb4, blk):
    n = y4.shape[0]
    return pl.pallas_call(
        functools.partial(_nodeC_body, n_nodes=float(n)),
        grid=(n // blk,),
        in_specs=[pl.BlockSpec((blk, 128), lambda i: (i, 0)),
                  pl.BlockSpec((2, 128), lambda i: (0, 0)),
                  pl.BlockSpec((2, 128), lambda i: (0, 0))],
        out_specs=pl.BlockSpec((blk, 128), lambda i: (i, 0)),
        out_shape=jax.ShapeDtypeStruct((n, 128), jnp.float32),
    )(y4, s4q4, gb4)


def kernel(x, edge_index, edge_attr, u, batch,
           W1, g1, b1, W2, g2, b2, W3, g3, b3, W4, g4, b4):
    n, d_node = x.shape
    e = edge_attr.shape[0]

    row = edge_index[0]
    col = edge_index[1]
    w1xt = jnp.transpose(W1[:, :d_node])
    w1at = jnp.transpose(W1[:, d_node:])
    w2t = jnp.transpose(W2)
    w3t = jnp.transpose(W3)
    w4t = jnp.transpose(W4)
    gb1 = jnp.stack([g1, b1])
    gb2 = jnp.stack([g2, b2])
    gb3 = jnp.stack([g3, b3])
    gb4 = jnp.stack([g4, b4])
    batch2 = batch.reshape(n, 1)

    n_pad = 10240
    p0 = _prep(x, w1xt)
    p0g, cnt2, aseg2 = _sc_gather(p0, col, edge_attr, e, n_pad)
    eas = _eastats(edge_attr, be=8000)
    s1q1 = _comb(p0, cnt2, aseg2, eas, w1at, n_pad)
    y2a, y2b, s2q2 = _pass2(p0g, edge_attr, w1at, w2t, s1q1, gb1, be=2560)
    z2, c2 = _sc_scatter(y2a, y2b, row, n_pad)
    z2a, z2b = z2[0:n], z2[n_pad:n_pad + n]
    c2a = c2[0:n]
    y3, s3q3 = _nodeA(x, z2a, z2b, c2a, u, batch2, s2q2, gb2, w3t, e,
                      blk=2000)
    y4, s4q4 = _nodeB(y3, s3q3, gb3, w4t, blk=2000)
    out = _nodeC(y4, s4q4, gb4, blk=2000)
    return out
